# baseline (speedup 1.0000x reference)
import functools

import jax
import jax.numpy as jnp
from jax import lax
from jax.experimental import pallas as pl
from jax.experimental.pallas import tpu as pltpu
from jax.experimental.pallas import tpu_sc as plsc

_NUM_WORKERS = 32
_CHUNK = 128
_NBUF = 8
_LOOK = 6


def _embedding_body(nchunks, chunk, nbuf, look, x_hbm, w_hbm, out_hbm, idx_v,
                    rows_v, *sems):
    gsems = sems[:nbuf]
    ssems = sems[nbuf:]
    b_per_w = nchunks * chunk
    wid = lax.axis_index("s") * 2 + lax.axis_index("c")
    base = pl.multiple_of(wid * b_per_w, b_per_w)

    def fire_gather(j, b):
        joff = pl.multiple_of(j * chunk, chunk)
        pltpu.async_copy(
            w_hbm.at[idx_v.at[pl.ds(joff, chunk)]], rows_v.at[b], gsems[b])

    def gather_wait(i, b):
        ioff = pl.multiple_of(i * chunk, chunk)
        pltpu.make_async_copy(
            w_hbm.at[idx_v.at[pl.ds(ioff, chunk)]], rows_v.at[b],
            gsems[b]).wait()

    def store_desc(i, b):
        ioff = pl.multiple_of(i * chunk, chunk)
        return pltpu.make_async_copy(
            rows_v.at[b], out_hbm.at[pl.ds(base + ioff, chunk)], ssems[b])

    pltpu.sync_copy(x_hbm.at[pl.ds(base, b_per_w)], idx_v)

    for b in range(look):
        fire_gather(b, b)

    def group(g, carry):
        for b in range(nbuf):
            i = g * nbuf + b
            gather_wait(i, b)
            store_desc(i, b).start()

            j = i + look
            bj = (b + look) % nbuf

            @pl.when(j < nchunks)
            def _():
                @pl.when(j >= nbuf)
                def _():
                    store_desc(j - nbuf, bj).wait()

                fire_gather(j, bj)

        return carry

    lax.fori_loop(0, nchunks // nbuf, group, 0)

    for b in range(nbuf):
        store_desc(nchunks - nbuf + b, b).wait()


@jax.jit
def kernel(x, weight):
    batch, hist = x.shape
    n = batch * hist
    dim = weight.shape[1]
    assert n % (_NUM_WORKERS * _CHUNK) == 0
    b_per_w = n // _NUM_WORKERS
    nchunks = b_per_w // _CHUNK

    idx = x.astype(jnp.int32).reshape(n)

    mesh = plsc.VectorSubcoreMesh(core_axis_name="c", subcore_axis_name="s")
    scratch = [
        pltpu.VMEM((b_per_w,), jnp.int32),
        pltpu.VMEM((_NBUF, _CHUNK, dim), jnp.float32),
    ] + [pltpu.SemaphoreType.DMA] * (2 * _NBUF)

    out = pl.kernel(
        functools.partial(_embedding_body, nchunks, _CHUNK, _NBUF, _LOOK),
        out_type=jax.ShapeDtypeStruct((n, dim), jnp.float32),
        mesh=mesh,
        scratch_types=scratch,
        compiler_params=pltpu.CompilerParams(use_tc_tiling_on_sc=False),
    )(idx, weight)
    return out.reshape(batch, hist, dim)
